# Initial kernel scaffold; baseline (speedup 1.0000x reference)
import jax
import jax.numpy as jnp
from jax.experimental import pallas as pl


def kernel(x, edge_index, edge_type, weight, bias_term, gate_weight, gate_bias):
    raise NotImplementedError("write your pallas kernel here")


# speedup vs baseline: 19.0771x; 19.0771x over previous
import functools

import jax
import jax.numpy as jnp
from jax import lax
from jax.experimental import pallas as pl
from jax.experimental.pallas import tpu as pltpu
from jax.experimental.pallas import tpu_sc as plsc

N = 50000
E = 1600000
IN_FEAT = 20
OUT_FEAT = 20
NUM_RELS = 500
NUM_BASES = 4
SUB = 5
HALF = 10

CHUNK = 128
NSUB = 16
K_CHUNKS = -(-E // (CHUNK * NSUB))
E_PAD = K_CHUNKS * CHUNK * NSUB
ROWS_PER_TILE = 3136
N_ACC = ROWS_PER_TILE * NSUB
STAGE = 392
XW = 32
AW = 16
WW = 56
BW = 16


def _splat(v):
    return jnp.full((16,), v, jnp.int32)


def _rgcn_sc_body(x_hbm, ed_hbm, w_hbm, b_hbm, gw_hbm, gb_hbm, part_hbm,
                  w_v, b_v, gw_v, gb_v, ed_v, xr_v, msg_v, stage_v, acc, sem):
    cid = lax.axis_index("c")
    sid = lax.axis_index("s")
    row0 = sid * ROWS_PER_TILE
    f0 = cid * HALF

    pltpu.sync_copy(w_hbm.at[cid], w_v)
    pltpu.sync_copy(b_hbm.at[cid], b_v)
    pltpu.sync_copy(gw_hbm, gw_v)
    pltpu.sync_copy(gb_hbm, gb_v)

    def _zero_stage(i, c):
        z = jnp.zeros((16,), jnp.float32)
        stage_v[i, pl.ds(0, 16)] = z
        return c
    lax.fori_loop(0, STAGE, _zero_stage, 0)

    def _zero_msg(i, c):
        msg_v[i, pl.ds(0, 16)] = jnp.zeros((16,), jnp.float32)
        return c
    lax.fori_loop(0, CHUNK, _zero_msg, 0)
    for s in range(ROWS_PER_TILE // STAGE):
        pltpu.sync_copy(stage_v, acc.at[pl.ds(row0 + s * STAGE, STAGE)])
    plsc.subcore_barrier()

    def chunk_body(k, c):
        base = (sid + NSUB * k) * CHUNK
        pltpu.sync_copy(ed_hbm.at[:, pl.ds(base, CHUNK)], ed_v)
        pltpu.async_copy(x_hbm.at[ed_v.at[0]], xr_v, sem).wait()

        def group_body(g, c2):
            e0 = g * 16
            rows = e0 + lax.iota(jnp.int32, 16)
            et = ed_v[2, pl.ds(e0, 16)]
            hs = [plsc.load_gather(xr_v, [rows, _splat(f) + f0])
                  for f in range(HALF)]
            ho = [plsc.load_gather(xr_v, [rows, _splat(f) + (HALF - f0)])
                  for f in range(HALF)]
            gacc = plsc.load_gather(gb_v, [et])
            for f in range(HALF):
                gacc = gacc + hs[f] * plsc.load_gather(
                    gw_v, [et, _splat(f) + f0])
                gacc = gacc + ho[f] * plsc.load_gather(
                    gw_v, [et, _splat(f) + (HALF - f0)])
            gate = 1.0 / (1.0 + jnp.exp(-gacc))
            for b in range(2):
                for j in range(SUB):
                    o = None
                    for i in range(SUB):
                        wv = plsc.load_gather(
                            w_v, [et, _splat(b * SUB * SUB + i * SUB + j)])
                        t = hs[b * SUB + i] * wv
                        o = t if o is None else o + t
                    o = o + plsc.load_gather(b_v, [et, _splat(b * SUB + j)])
                    o = gate * o
                    plsc.store_scatter(msg_v, [rows, _splat(b * SUB + j)], o)
            return c2
        lax.fori_loop(0, CHUNK // 16, group_body, 0)
        pltpu.sync_copy(msg_v, acc.at[ed_v.at[1]], add=True)
        return c
    lax.fori_loop(0, K_CHUNKS, chunk_body, 0)

    plsc.subcore_barrier()
    for s in range(ROWS_PER_TILE // STAGE):
        r = row0 + s * STAGE
        pltpu.sync_copy(acc.at[pl.ds(r, STAGE)], stage_v)
        pltpu.sync_copy(stage_v, part_hbm.at[cid, pl.ds(r, STAGE)])


_rgcn_sc = functools.partial(
    pl.kernel,
    out_type=jax.ShapeDtypeStruct((2, N_ACC, AW), jnp.float32),
    mesh=plsc.VectorSubcoreMesh(core_axis_name="c", subcore_axis_name="s"),
    scratch_types=[
        pltpu.VMEM((NUM_RELS, WW), jnp.float32),
        pltpu.VMEM((NUM_RELS, BW), jnp.float32),
        pltpu.VMEM((NUM_RELS, IN_FEAT), jnp.float32),
        pltpu.VMEM((NUM_RELS,), jnp.float32),
        pltpu.VMEM((3, CHUNK), jnp.int32),
        pltpu.VMEM((CHUNK, XW), jnp.float32),
        pltpu.VMEM((CHUNK, AW), jnp.float32),
        pltpu.VMEM((STAGE, AW), jnp.float32),
        pltpu.VMEM_SHARED((N_ACC, AW), jnp.float32),
        pltpu.SemaphoreType.DMA,
    ],
    compiler_params=pltpu.CompilerParams(
        needs_layout_passes=False, use_tc_tiling_on_sc=False),
)(_rgcn_sc_body)


def _cat_body(p0_ref, p1_ref, o_ref):
    o_ref[:, :HALF] = p0_ref[:, :HALF]
    o_ref[:, HALF:OUT_FEAT] = p1_ref[:, :HALF]


def _cat_partials(part):
    blk = 400
    return pl.pallas_call(
        _cat_body,
        grid=(N // blk,),
        in_specs=[
            pl.BlockSpec((blk, AW), lambda i: (i, 0)),
            pl.BlockSpec((blk, AW), lambda i: (i, 0)),
        ],
        out_specs=pl.BlockSpec((blk, OUT_FEAT), lambda i: (i, 0)),
        out_shape=jax.ShapeDtypeStruct((N, OUT_FEAT), jnp.float32),
    )(part[0, :N], part[1, :N])


@jax.jit
def kernel(x, edge_index, edge_type, weight, bias_term, gate_weight, gate_bias):
    pad = E_PAD - E
    src = jnp.pad(edge_index[0], (0, pad))
    dst = jnp.pad(edge_index[1], (0, pad), constant_values=N)
    et = jnp.pad(edge_type, (0, pad))
    ed = jnp.stack([src, dst, et])
    x_pad = jnp.pad(x, ((0, 0), (0, XW - IN_FEAT)))
    w4 = weight.reshape(NUM_RELS, NUM_BASES, SUB * SUB)
    w2 = jnp.stack([
        w4[:, 0:2].reshape(NUM_RELS, 2 * SUB * SUB),
        w4[:, 2:4].reshape(NUM_RELS, 2 * SUB * SUB),
    ])
    w2 = jnp.pad(w2, ((0, 0), (0, 0), (0, WW - 2 * SUB * SUB)))
    b2 = jnp.stack([bias_term[:, :HALF], bias_term[:, HALF:]])
    b2 = jnp.pad(b2, ((0, 0), (0, 0), (0, BW - HALF)))
    part = _rgcn_sc(x_pad, ed, w2, b2,
                    gate_weight.reshape(NUM_RELS, IN_FEAT),
                    gate_bias.reshape(NUM_RELS))
    return _cat_partials(part)
